# baseline (speedup 1.0000x reference)
import functools

import jax
import jax.numpy as jnp
from jax import lax
from jax.experimental import pallas as pl
from jax.experimental.pallas import tpu as pltpu
from jax.experimental.pallas import tpu_sc as plsc

EPS = 1e-8


def _vb_body(x_ref, w1mu_ref, w1ls_ref, w2mu_ref, w2ls_ref, out_ref):
    x = x_ref[...]
    d1 = w1mu_ref.shape[1]
    d2 = w2mu_ref.shape[1]
    scale = (2.0 / d1) ** 0.5
    w1mu = scale * w1mu_ref[...]
    sig21 = (scale * scale) * jnp.exp(w1ls_ref[...])
    w2mu = w2mu_ref[...]
    sig22 = jnp.exp(w2ls_ref[...])

    z = jnp.zeros_like(w1mu)
    wa = jnp.concatenate(
        [jnp.concatenate([w1mu, z], axis=1),
         jnp.concatenate([z, sig21], axis=1)], axis=0)
    mv1 = jnp.dot(jnp.concatenate([x, x * x], axis=1), wa,
                  preferred_element_type=jnp.float32)
    m1 = jnp.maximum(mv1[:, :d1], 0.0)
    v1 = mv1[:, d1:]

    z2 = jnp.zeros_like(w2mu)
    wb = jnp.concatenate(
        [jnp.concatenate([w2mu, z2], axis=1),
         jnp.concatenate([z2, sig22], axis=1),
         jnp.concatenate([z2, w2mu * w2mu], axis=1)], axis=0)
    mv2 = jnp.dot(jnp.concatenate([m1, m1 * m1 + v1, v1], axis=1), wb,
                  preferred_element_type=jnp.float32)
    m2 = mv2[:, :d2]
    v2 = mv2[:, d2:]

    p = 1.0 / (v2 + EPS)
    out_ref[...] = jnp.concatenate([p, p * m2], axis=1)


def _vb_layers(X, W1_mu, W1_logsig2, W2_mu, W2_logsig2, block_rows):
    n, d0 = X.shape
    d1 = W1_mu.shape[1]
    d2 = W2_mu.shape[1]
    grid = n // block_rows
    return pl.pallas_call(
        _vb_body,
        grid=(grid,),
        in_specs=[
            pl.BlockSpec((block_rows, d0), lambda i: (i, 0)),
            pl.BlockSpec((d0, d1), lambda i: (0, 0)),
            pl.BlockSpec((d0, d1), lambda i: (0, 0)),
            pl.BlockSpec((d1, d2), lambda i: (0, 0)),
            pl.BlockSpec((d1, d2), lambda i: (0, 0)),
        ],
        out_specs=pl.BlockSpec((block_rows, 2 * d2), lambda i: (i, 0)),
        out_shape=jax.ShapeDtypeStruct((n, 2 * d2), jnp.float32),
    )(X, W1_mu, W1_logsig2, W2_mu, W2_logsig2)


def _make_seg_sum(n, num_seg_pad, width, chunk):
    info = plsc.get_sparse_core_info()
    nc, ns = info.num_cores, info.num_subcores
    nw = nc * ns
    rows_per_tile = n // nw
    n_chunks = rows_per_tile // chunk
    segs_per_tile = num_seg_pad // ns

    mesh = plsc.VectorSubcoreMesh(core_axis_name="c", subcore_axis_name="s")

    @functools.partial(
        pl.kernel,
        out_type=jax.ShapeDtypeStruct((nc, num_seg_pad, width), jnp.float32),
        mesh=mesh,
        scratch_types=[
            pltpu.VMEM((n_chunks, chunk), jnp.int32),
            pltpu.VMEM((chunk, width), jnp.float32),
            pltpu.VMEM((chunk, width), jnp.float32),
            pltpu.VMEM_SHARED((num_seg_pad, width), jnp.float32),
            pltpu.SemaphoreType.DMA,
            pltpu.SemaphoreType.DMA,
        ],
    )
    def seg_sum(pw_hbm, idx_hbm, zeros_hbm, part_hbm,
                idx_v, buf0, buf1, acc_sh, sem0, sem1):
        cid = lax.axis_index("c")
        sid = lax.axis_index("s")
        wid = sid * nc + cid
        base = wid * rows_per_tile
        bufs = (buf0, buf1)
        sems = (sem0, sem1)

        pltpu.sync_copy(idx_hbm.at[wid], idx_v)
        pltpu.sync_copy(
            zeros_hbm.at[pl.ds(sid * segs_per_tile, segs_per_tile)],
            acc_sh.at[pl.ds(sid * segs_per_tile, segs_per_tile)],
        )
        plsc.subcore_barrier()

        def issue(g, b):
            pltpu.async_copy(pw_hbm.at[pl.ds(base + g * chunk, chunk)],
                             bufs[b], sems[b])

        for b in range(2):
            issue(b, b)

        def body(j, carry):
            for b in range(2):
                g = 2 * j + b
                pltpu.make_async_copy(pw_hbm.at[pl.ds(0, chunk)],
                                      bufs[b], sems[b]).wait()
                pltpu.sync_copy(bufs[b], acc_sh.at[idx_v.at[g]], add=True)
                issue(jnp.minimum(g + 2, n_chunks - 1), b)
            return carry

        lax.fori_loop(0, (n_chunks - 1) // 2, body, 0)

        gl = n_chunks - 1
        pltpu.make_async_copy(pw_hbm.at[pl.ds(0, chunk)], bufs[0], sems[0]).wait()
        pltpu.sync_copy(bufs[0], acc_sh.at[idx_v.at[gl]], add=True)
        pltpu.make_async_copy(pw_hbm.at[pl.ds(0, chunk)], bufs[1], sems[1]).wait()
        plsc.subcore_barrier()

        pltpu.sync_copy(
            acc_sh.at[pl.ds(sid * segs_per_tile, segs_per_tile)],
            part_hbm.at[cid, pl.ds(sid * segs_per_tile, segs_per_tile)],
        )

    return seg_sum


def _fin_body(part_ref, means_ref, vars_ref):
    num_seg, d2 = means_ref.shape
    s = part_ref[0, :num_seg, :] + part_ref[1, :num_seg, :]
    w = s[:, :d2] + EPS
    var = 1.0 / w
    means_ref[...] = s[:, d2:] * var
    vars_ref[...] = var


def _finalize(part, num_seg, d2):
    return pl.pallas_call(
        _fin_body,
        out_shape=(
            jax.ShapeDtypeStruct((num_seg, d2), jnp.float32),
            jax.ShapeDtypeStruct((num_seg, d2), jnp.float32),
        ),
    )(part)


def kernel(X, W1_mu, W1_logsig2, W2_mu, W2_logsig2, X_idx):
    n = X.shape[0]
    d2 = W2_mu.shape[1]
    num_seg = 10000
    num_seg_pad = 10240
    width = 2 * d2

    chunk = 80
    info = plsc.get_sparse_core_info()
    nw = info.num_cores * info.num_subcores
    n_chunks = n // (nw * chunk)
    assert n_chunks % 2 == 1
    pw = _vb_layers(X, W1_mu, W1_logsig2, W2_mu, W2_logsig2, block_rows=2000)
    zeros = jnp.zeros((num_seg_pad, width), jnp.float32)
    idx3d = X_idx.reshape(nw, n_chunks, chunk)
    part = _make_seg_sum(n, num_seg_pad, width, chunk=chunk)(pw, idx3d, zeros)
    means, vars_ = _finalize(part, num_seg, d2)
    return means, vars_


# speedup vs baseline: 1.1210x; 1.1210x over previous
import functools

import jax
import jax.numpy as jnp
from jax import lax
from jax.experimental import pallas as pl
from jax.experimental.pallas import tpu as pltpu
from jax.experimental.pallas import tpu_sc as plsc

EPS = 1e-8


def _vb_body(x_ref, w1mu_ref, w1ls_ref, w2mu_ref, w2ls_ref, out_ref):
    x = x_ref[...]
    d1 = w1mu_ref.shape[1]
    d2 = w2mu_ref.shape[1]
    scale = (2.0 / d1) ** 0.5
    w1mu = scale * w1mu_ref[...]
    sig21 = (scale * scale) * jnp.exp(w1ls_ref[...])
    w2mu = w2mu_ref[...]
    sig22 = jnp.exp(w2ls_ref[...])

    z = jnp.zeros_like(w1mu)
    wa = jnp.concatenate(
        [jnp.concatenate([w1mu, z], axis=1),
         jnp.concatenate([z, sig21], axis=1)], axis=0)
    mv1 = jnp.dot(jnp.concatenate([x, x * x], axis=1), wa,
                  preferred_element_type=jnp.float32)
    m1 = jnp.maximum(mv1[:, :d1], 0.0)
    v1 = mv1[:, d1:]

    z2 = jnp.zeros_like(w2mu)
    wb = jnp.concatenate(
        [jnp.concatenate([w2mu, z2], axis=1),
         jnp.concatenate([z2, sig22], axis=1),
         jnp.concatenate([z2, w2mu * w2mu], axis=1)], axis=0)
    mv2 = jnp.dot(jnp.concatenate([m1, m1 * m1 + v1, v1], axis=1), wb,
                  preferred_element_type=jnp.float32)
    m2 = mv2[:, :d2]
    v2 = mv2[:, d2:]

    p = 1.0 / (v2 + EPS)
    out_ref[...] = jnp.concatenate([p, p * m2], axis=1)


def _vb_layers(X, W1_mu, W1_logsig2, W2_mu, W2_logsig2, block_rows,
               row_block0, n_rows):
    d0 = X.shape[1]
    d1 = W1_mu.shape[1]
    d2 = W2_mu.shape[1]
    grid = n_rows // block_rows
    return pl.pallas_call(
        _vb_body,
        grid=(grid,),
        in_specs=[
            pl.BlockSpec((block_rows, d0), lambda i: (row_block0 + i, 0)),
            pl.BlockSpec((d0, d1), lambda i: (0, 0)),
            pl.BlockSpec((d0, d1), lambda i: (0, 0)),
            pl.BlockSpec((d1, d2), lambda i: (0, 0)),
            pl.BlockSpec((d1, d2), lambda i: (0, 0)),
        ],
        out_specs=pl.BlockSpec((block_rows, 2 * d2), lambda i: (i, 0)),
        out_shape=jax.ShapeDtypeStruct((n_rows, 2 * d2), jnp.float32),
    )(X, W1_mu, W1_logsig2, W2_mu, W2_logsig2)


def _make_seg_sum(n, num_seg_pad, width, chunk):
    info = plsc.get_sparse_core_info()
    nc, ns = info.num_cores, info.num_subcores
    nw = nc * ns
    rows_per_tile = n // nw
    n_chunks = rows_per_tile // chunk
    segs_per_tile = num_seg_pad // ns

    mesh = plsc.VectorSubcoreMesh(core_axis_name="c", subcore_axis_name="s")

    @functools.partial(
        pl.kernel,
        out_type=jax.ShapeDtypeStruct((nc, num_seg_pad, width), jnp.float32),
        mesh=mesh,
        scratch_types=[
            pltpu.VMEM((n_chunks, chunk), jnp.int32),
            pltpu.VMEM((chunk, width), jnp.float32),
            pltpu.VMEM((chunk, width), jnp.float32),
            pltpu.VMEM_SHARED((num_seg_pad, width), jnp.float32),
            pltpu.SemaphoreType.DMA,
            pltpu.SemaphoreType.DMA,
        ],
    )
    def seg_sum(pw_hbm, idx_hbm, zeros_hbm, part_hbm,
                idx_v, buf0, buf1, acc_sh, sem0, sem1):
        cid = lax.axis_index("c")
        sid = lax.axis_index("s")
        wid = sid * nc + cid
        base = wid * rows_per_tile
        bufs = (buf0, buf1)
        sems = (sem0, sem1)

        pltpu.sync_copy(idx_hbm.at[wid], idx_v)
        pltpu.sync_copy(
            zeros_hbm.at[pl.ds(sid * segs_per_tile, segs_per_tile)],
            acc_sh.at[pl.ds(sid * segs_per_tile, segs_per_tile)],
        )
        plsc.subcore_barrier()

        def issue(g, b):
            pltpu.async_copy(pw_hbm.at[pl.ds(base + g * chunk, chunk)],
                             bufs[b], sems[b])

        for b in range(2):
            issue(b, b)

        def body(j, carry):
            for b in range(2):
                g = 2 * j + b
                pltpu.make_async_copy(pw_hbm.at[pl.ds(0, chunk)],
                                      bufs[b], sems[b]).wait()
                pltpu.sync_copy(bufs[b], acc_sh.at[idx_v.at[g]], add=True)
                issue(jnp.minimum(g + 2, n_chunks - 1), b)
            return carry

        lax.fori_loop(0, (n_chunks - 1) // 2, body, 0)

        gl = n_chunks - 1
        pltpu.make_async_copy(pw_hbm.at[pl.ds(0, chunk)], bufs[0], sems[0]).wait()
        pltpu.sync_copy(bufs[0], acc_sh.at[idx_v.at[gl]], add=True)
        pltpu.make_async_copy(pw_hbm.at[pl.ds(0, chunk)], bufs[1], sems[1]).wait()
        plsc.subcore_barrier()

        pltpu.sync_copy(
            acc_sh.at[pl.ds(sid * segs_per_tile, segs_per_tile)],
            part_hbm.at[cid, pl.ds(sid * segs_per_tile, segs_per_tile)],
        )

    return seg_sum


def _fin_body(*refs):
    means_ref, vars_ref = refs[-2], refs[-1]
    d2 = means_ref.shape[1]
    s = refs[0][0] + refs[0][1]
    for r in refs[1:-2]:
        s = s + r[0] + r[1]
    w = s[:, :d2] + EPS
    var = 1.0 / w
    means_ref[...] = s[:, d2:] * var
    vars_ref[...] = var


def _finalize(parts, num_seg_pad, d2):
    width = 2 * d2
    bs = 1280
    grid = num_seg_pad // bs
    return pl.pallas_call(
        _fin_body,
        grid=(grid,),
        in_specs=[pl.BlockSpec((2, bs, width), lambda i: (0, i, 0))
                  for _ in parts],
        out_specs=(pl.BlockSpec((bs, d2), lambda i: (i, 0)),
                   pl.BlockSpec((bs, d2), lambda i: (i, 0))),
        out_shape=(
            jax.ShapeDtypeStruct((num_seg_pad, d2), jnp.float32),
            jax.ShapeDtypeStruct((num_seg_pad, d2), jnp.float32),
        ),
    )(*parts)


def kernel(X, W1_mu, W1_logsig2, W2_mu, W2_logsig2, X_idx):
    n = X.shape[0]
    d2 = W2_mu.shape[1]
    num_seg = 10000
    num_seg_pad = 10240
    width = 2 * d2

    nchunk = 5
    chunk = 80
    block_rows = 2000
    info = plsc.get_sparse_core_info()
    nw = info.num_cores * info.num_subcores
    n_c = n // nchunk
    n_chunks = n_c // (nw * chunk)
    assert n_chunks % 2 == 1
    assert n_c % block_rows == 0

    zeros = jnp.zeros((num_seg_pad, width), jnp.float32)
    idx4d = X_idx.reshape(nchunk, nw, n_chunks, chunk)
    seg_sum = _make_seg_sum(n_c, num_seg_pad, width, chunk=chunk)

    parts = []
    for c in range(nchunk):
        pw = _vb_layers(X, W1_mu, W1_logsig2, W2_mu, W2_logsig2,
                        block_rows=block_rows,
                        row_block0=c * (n_c // block_rows), n_rows=n_c)
        parts.append(seg_sum(pw, idx4d[c], zeros))

    means_p, vars_p = _finalize(parts, num_seg_pad, d2)
    return means_p[:num_seg], vars_p[:num_seg]


# speedup vs baseline: 1.1497x; 1.0256x over previous
import functools

import jax
import jax.numpy as jnp
from jax import lax
from jax.experimental import pallas as pl
from jax.experimental.pallas import tpu as pltpu
from jax.experimental.pallas import tpu_sc as plsc

EPS = 1e-8


def _vb_body(x_ref, w1mu_ref, w1ls_ref, w2mu_ref, w2ls_ref, out_ref):
    x = x_ref[...]
    d1 = w1mu_ref.shape[1]
    d2 = w2mu_ref.shape[1]
    scale = (2.0 / d1) ** 0.5
    w1mu = scale * w1mu_ref[...]
    sig21 = (scale * scale) * jnp.exp(w1ls_ref[...])
    w2mu = w2mu_ref[...]
    sig22 = jnp.exp(w2ls_ref[...])

    z = jnp.zeros_like(w1mu)
    wa = jnp.concatenate(
        [jnp.concatenate([w1mu, z], axis=1),
         jnp.concatenate([z, sig21], axis=1)], axis=0)
    mv1 = jnp.dot(jnp.concatenate([x, x * x], axis=1), wa,
                  preferred_element_type=jnp.float32)
    m1 = jnp.maximum(mv1[:, :d1], 0.0)
    v1 = mv1[:, d1:]

    z2 = jnp.zeros_like(w2mu)
    wb = jnp.concatenate(
        [jnp.concatenate([w2mu, z2], axis=1),
         jnp.concatenate([z2, sig22], axis=1),
         jnp.concatenate([z2, w2mu * w2mu], axis=1)], axis=0)
    mv2 = jnp.dot(jnp.concatenate([m1, m1 * m1 + v1, v1], axis=1), wb,
                  preferred_element_type=jnp.float32)
    m2 = mv2[:, :d2]
    v2 = mv2[:, d2:]

    p = 1.0 / (v2 + EPS)
    out_ref[...] = jnp.concatenate([p, p * m2], axis=1)


def _vb_layers(X, W1_mu, W1_logsig2, W2_mu, W2_logsig2, block_rows,
               row_block0, n_rows):
    d0 = X.shape[1]
    d1 = W1_mu.shape[1]
    d2 = W2_mu.shape[1]
    grid = n_rows // block_rows
    return pl.pallas_call(
        _vb_body,
        grid=(grid,),
        in_specs=[
            pl.BlockSpec((block_rows, d0), lambda i: (row_block0 + i, 0)),
            pl.BlockSpec((d0, d1), lambda i: (0, 0)),
            pl.BlockSpec((d0, d1), lambda i: (0, 0)),
            pl.BlockSpec((d1, d2), lambda i: (0, 0)),
            pl.BlockSpec((d1, d2), lambda i: (0, 0)),
        ],
        out_specs=pl.BlockSpec((block_rows, 2 * d2), lambda i: (i, 0)),
        out_shape=jax.ShapeDtypeStruct((n_rows, 2 * d2), jnp.float32),
    )(X, W1_mu, W1_logsig2, W2_mu, W2_logsig2)


def _make_seg_sum(n, num_seg_pad, width, chunk):
    info = plsc.get_sparse_core_info()
    nc, ns = info.num_cores, info.num_subcores
    nw = nc * ns
    rows_per_tile = n // nw
    n_chunks = rows_per_tile // chunk
    segs_per_tile = num_seg_pad // ns

    mesh = plsc.VectorSubcoreMesh(core_axis_name="c", subcore_axis_name="s")

    @functools.partial(
        pl.kernel,
        out_type=jax.ShapeDtypeStruct((nc, num_seg_pad, width), jnp.float32),
        mesh=mesh,
        scratch_types=[
            pltpu.VMEM((n_chunks, chunk), jnp.int32),
            pltpu.VMEM((chunk, width), jnp.float32),
            pltpu.VMEM((chunk, width), jnp.float32),
            pltpu.VMEM((chunk, width), jnp.float32),
            pltpu.VMEM_SHARED((num_seg_pad, width), jnp.float32),
            pltpu.SemaphoreType.DMA,
            pltpu.SemaphoreType.DMA,
        ],
    )
    def seg_sum(pw_hbm, idx_hbm, part_hbm,
                idx_v, buf0, buf1, zbuf, acc_sh, sem0, sem1):
        cid = lax.axis_index("c")
        sid = lax.axis_index("s")
        wid = sid * nc + cid
        base = wid * rows_per_tile
        bufs = (buf0, buf1)
        sems = (sem0, sem1)

        def issue(g, b):
            pltpu.async_copy(pw_hbm.at[pl.ds(base + g * chunk, chunk)],
                             bufs[b], sems[b])

        for b in range(2):
            issue(b, b)

        pltpu.sync_copy(idx_hbm.at[wid], idx_v)

        z16 = jnp.zeros((16,), jnp.float32)

        def zrow(i, c):
            for j in range(width // 16):
                zbuf[i, pl.ds(j * 16, 16)] = z16
            return c

        lax.fori_loop(0, chunk, zrow, 0)
        for k in range(segs_per_tile // chunk):
            pltpu.sync_copy(
                zbuf,
                acc_sh.at[pl.ds(sid * segs_per_tile + k * chunk, chunk)])
        plsc.subcore_barrier()


        def body(j, carry):
            for b in range(2):
                g = 2 * j + b
                pltpu.make_async_copy(pw_hbm.at[pl.ds(0, chunk)],
                                      bufs[b], sems[b]).wait()
                pltpu.sync_copy(bufs[b], acc_sh.at[idx_v.at[g]], add=True)
                issue(jnp.minimum(g + 2, n_chunks - 1), b)
            return carry

        lax.fori_loop(0, (n_chunks - 1) // 2, body, 0)

        gl = n_chunks - 1
        pltpu.make_async_copy(pw_hbm.at[pl.ds(0, chunk)], bufs[0], sems[0]).wait()
        pltpu.sync_copy(bufs[0], acc_sh.at[idx_v.at[gl]], add=True)
        pltpu.make_async_copy(pw_hbm.at[pl.ds(0, chunk)], bufs[1], sems[1]).wait()
        plsc.subcore_barrier()

        pltpu.sync_copy(
            acc_sh.at[pl.ds(sid * segs_per_tile, segs_per_tile)],
            part_hbm.at[cid, pl.ds(sid * segs_per_tile, segs_per_tile)],
        )

    return seg_sum


def _fin_body(*refs):
    means_ref, vars_ref = refs[-2], refs[-1]
    d2 = means_ref.shape[1]
    s = refs[0][0] + refs[0][1]
    for r in refs[1:-2]:
        s = s + r[0] + r[1]
    w = s[:, :d2] + EPS
    var = 1.0 / w
    means_ref[...] = s[:, d2:] * var
    vars_ref[...] = var


def _finalize(parts, num_seg_pad, d2):
    width = 2 * d2
    bs = 1280
    grid = num_seg_pad // bs
    return pl.pallas_call(
        _fin_body,
        grid=(grid,),
        in_specs=[pl.BlockSpec((2, bs, width), lambda i: (0, i, 0))
                  for _ in parts],
        out_specs=(pl.BlockSpec((bs, d2), lambda i: (i, 0)),
                   pl.BlockSpec((bs, d2), lambda i: (i, 0))),
        out_shape=(
            jax.ShapeDtypeStruct((num_seg_pad, d2), jnp.float32),
            jax.ShapeDtypeStruct((num_seg_pad, d2), jnp.float32),
        ),
    )(*parts)


def kernel(X, W1_mu, W1_logsig2, W2_mu, W2_logsig2, X_idx):
    n = X.shape[0]
    d2 = W2_mu.shape[1]
    num_seg = 10000
    num_seg_pad = 10240
    width = 2 * d2

    nchunk = 5
    chunk = 80
    block_rows = 2000
    info = plsc.get_sparse_core_info()
    nw = info.num_cores * info.num_subcores
    n_c = n // nchunk
    n_chunks = n_c // (nw * chunk)
    assert n_chunks % 2 == 1
    assert n_c % block_rows == 0

    idx4d = X_idx.reshape(nchunk, nw, n_chunks, chunk)
    seg_sum = _make_seg_sum(n_c, num_seg_pad, width, chunk=chunk)

    parts = []
    for c in range(nchunk):
        pw = _vb_layers(X, W1_mu, W1_logsig2, W2_mu, W2_logsig2,
                        block_rows=block_rows,
                        row_block0=c * (n_c // block_rows), n_rows=n_c)
        parts.append(seg_sum(pw, idx4d[c]))

    means_p, vars_p = _finalize(parts, num_seg_pad, d2)
    return means_p[:num_seg], vars_p[:num_seg]


# speedup vs baseline: 1.1620x; 1.0107x over previous
import functools

import jax
import jax.numpy as jnp
from jax import lax
from jax.experimental import pallas as pl
from jax.experimental.pallas import tpu as pltpu
from jax.experimental.pallas import tpu_sc as plsc

EPS = 1e-8


def _vb_body(x_ref, w1mu_ref, w1ls_ref, w2mu_ref, w2ls_ref, out_ref):
    x = x_ref[...]
    d1 = w1mu_ref.shape[1]
    d2 = w2mu_ref.shape[1]
    scale = (2.0 / d1) ** 0.5
    w1mu = scale * w1mu_ref[...]
    sig21 = (scale * scale) * jnp.exp(w1ls_ref[...])
    w2mu = w2mu_ref[...]
    sig22 = jnp.exp(w2ls_ref[...])

    dot = functools.partial(jnp.dot, preferred_element_type=jnp.float32)
    m1 = jnp.maximum(dot(x, w1mu), 0.0)
    v1 = dot(x * x, sig21)
    m2 = dot(m1, w2mu)
    v2 = dot(m1 * m1 + v1, sig22) + dot(v1, w2mu * w2mu)

    p = 1.0 / (v2 + EPS)
    out_ref[:, :d2] = p
    out_ref[:, d2:] = p * m2


def _vb_layers(X, W1_mu, W1_logsig2, W2_mu, W2_logsig2, block_rows,
               row_block0, n_rows):
    d0 = X.shape[1]
    d1 = W1_mu.shape[1]
    d2 = W2_mu.shape[1]
    grid = n_rows // block_rows
    return pl.pallas_call(
        _vb_body,
        grid=(grid,),
        in_specs=[
            pl.BlockSpec((block_rows, d0), lambda i: (row_block0 + i, 0)),
            pl.BlockSpec((d0, d1), lambda i: (0, 0)),
            pl.BlockSpec((d0, d1), lambda i: (0, 0)),
            pl.BlockSpec((d1, d2), lambda i: (0, 0)),
            pl.BlockSpec((d1, d2), lambda i: (0, 0)),
        ],
        out_specs=pl.BlockSpec((block_rows, 2 * d2), lambda i: (i, 0)),
        out_shape=jax.ShapeDtypeStruct((n_rows, 2 * d2), jnp.float32),
    )(X, W1_mu, W1_logsig2, W2_mu, W2_logsig2)


def _make_seg_sum(n, num_seg_pad, width, chunk):
    info = plsc.get_sparse_core_info()
    nc, ns = info.num_cores, info.num_subcores
    nw = nc * ns
    rows_per_tile = n // nw
    n_chunks = rows_per_tile // chunk
    segs_per_tile = num_seg_pad // ns

    mesh = plsc.VectorSubcoreMesh(core_axis_name="c", subcore_axis_name="s")

    @functools.partial(
        pl.kernel,
        out_type=jax.ShapeDtypeStruct((nc, num_seg_pad, width), jnp.float32),
        mesh=mesh,
        scratch_types=[
            pltpu.VMEM((n_chunks, chunk), jnp.int32),
            pltpu.VMEM((chunk, width), jnp.float32),
            pltpu.VMEM((chunk, width), jnp.float32),
            pltpu.VMEM((chunk, width), jnp.float32),
            pltpu.VMEM_SHARED((num_seg_pad, width), jnp.float32),
            pltpu.SemaphoreType.DMA,
            pltpu.SemaphoreType.DMA,
        ],
    )
    def seg_sum(pw_hbm, idx_hbm, part_hbm,
                idx_v, buf0, buf1, zbuf, acc_sh, sem0, sem1):
        cid = lax.axis_index("c")
        sid = lax.axis_index("s")
        wid = sid * nc + cid
        base = wid * rows_per_tile
        bufs = (buf0, buf1)
        sems = (sem0, sem1)

        def issue(g, b):
            pltpu.async_copy(pw_hbm.at[pl.ds(base + g * chunk, chunk)],
                             bufs[b], sems[b])

        for b in range(2):
            issue(b, b)

        pltpu.sync_copy(idx_hbm.at[wid], idx_v)

        z16 = jnp.zeros((16,), jnp.float32)

        def zrow(i, c):
            for j in range(width // 16):
                zbuf[i, pl.ds(j * 16, 16)] = z16
            return c

        lax.fori_loop(0, chunk, zrow, 0)
        for k in range(segs_per_tile // chunk):
            pltpu.sync_copy(
                zbuf,
                acc_sh.at[pl.ds(sid * segs_per_tile + k * chunk, chunk)])
        plsc.subcore_barrier()


        def body(j, carry):
            for b in range(2):
                g = 2 * j + b
                pltpu.make_async_copy(pw_hbm.at[pl.ds(0, chunk)],
                                      bufs[b], sems[b]).wait()
                pltpu.sync_copy(bufs[b], acc_sh.at[idx_v.at[g]], add=True)
                issue(jnp.minimum(g + 2, n_chunks - 1), b)
            return carry

        lax.fori_loop(0, (n_chunks - 1) // 2, body, 0)

        gl = n_chunks - 1
        pltpu.make_async_copy(pw_hbm.at[pl.ds(0, chunk)], bufs[0], sems[0]).wait()
        pltpu.sync_copy(bufs[0], acc_sh.at[idx_v.at[gl]], add=True)
        pltpu.make_async_copy(pw_hbm.at[pl.ds(0, chunk)], bufs[1], sems[1]).wait()
        plsc.subcore_barrier()

        pltpu.sync_copy(
            acc_sh.at[pl.ds(sid * segs_per_tile, segs_per_tile)],
            part_hbm.at[cid, pl.ds(sid * segs_per_tile, segs_per_tile)],
        )

    return seg_sum


def _fin_body(*refs):
    means_ref, vars_ref = refs[-2], refs[-1]
    d2 = means_ref.shape[1]
    s = refs[0][0] + refs[0][1]
    for r in refs[1:-2]:
        s = s + r[0] + r[1]
    w = s[:, :d2] + EPS
    var = 1.0 / w
    means_ref[...] = s[:, d2:] * var
    vars_ref[...] = var


def _finalize(parts, num_seg_pad, d2):
    width = 2 * d2
    bs = 1280
    grid = num_seg_pad // bs
    return pl.pallas_call(
        _fin_body,
        grid=(grid,),
        in_specs=[pl.BlockSpec((2, bs, width), lambda i: (0, i, 0))
                  for _ in parts],
        out_specs=(pl.BlockSpec((bs, d2), lambda i: (i, 0)),
                   pl.BlockSpec((bs, d2), lambda i: (i, 0))),
        out_shape=(
            jax.ShapeDtypeStruct((num_seg_pad, d2), jnp.float32),
            jax.ShapeDtypeStruct((num_seg_pad, d2), jnp.float32),
        ),
    )(*parts)


def kernel(X, W1_mu, W1_logsig2, W2_mu, W2_logsig2, X_idx):
    n = X.shape[0]
    d2 = W2_mu.shape[1]
    num_seg = 10000
    num_seg_pad = 10240
    width = 2 * d2

    nchunk = 5
    chunk = 80
    block_rows = 2000
    info = plsc.get_sparse_core_info()
    nw = info.num_cores * info.num_subcores
    n_c = n // nchunk
    n_chunks = n_c // (nw * chunk)
    assert n_chunks % 2 == 1
    assert n_c % block_rows == 0

    idx4d = X_idx.reshape(nchunk, nw, n_chunks, chunk)
    seg_sum = _make_seg_sum(n_c, num_seg_pad, width, chunk=chunk)

    parts = []
    for c in range(nchunk):
        pw = _vb_layers(X, W1_mu, W1_logsig2, W2_mu, W2_logsig2,
                        block_rows=block_rows,
                        row_block0=c * (n_c // block_rows), n_rows=n_c)
        parts.append(seg_sum(pw, idx4d[c]))

    means_p, vars_p = _finalize(parts, num_seg_pad, d2)
    return means_p[:num_seg], vars_p[:num_seg]


# speedup vs baseline: 1.1659x; 1.0033x over previous
import functools

import jax
import jax.numpy as jnp
from jax import lax
from jax.experimental import pallas as pl
from jax.experimental.pallas import tpu as pltpu
from jax.experimental.pallas import tpu_sc as plsc

EPS = 1e-8


def _vb_body(x_ref, w1mu_ref, w1ls_ref, w2mu_ref, w2ls_ref, out_ref):
    x = x_ref[...]
    d1 = w1mu_ref.shape[1]
    d2 = w2mu_ref.shape[1]
    scale = (2.0 / d1) ** 0.5
    w1mu = scale * w1mu_ref[...]
    sig21 = (scale * scale) * jnp.exp(w1ls_ref[...])
    w2mu = w2mu_ref[...]
    sig22 = jnp.exp(w2ls_ref[...])

    dot = functools.partial(jnp.dot, preferred_element_type=jnp.float32)
    m1 = jnp.maximum(dot(x, w1mu), 0.0)
    v1 = dot(x * x, sig21)
    m2 = dot(m1, w2mu)
    v2 = dot(m1 * m1 + v1, sig22) + dot(v1, w2mu * w2mu)

    p = 1.0 / (v2 + EPS)
    out_ref[:, :d2] = p
    out_ref[:, d2:] = p * m2


def _vb_layers(X, W1_mu, W1_logsig2, W2_mu, W2_logsig2, block_rows,
               row_block0, n_rows):
    d0 = X.shape[1]
    d1 = W1_mu.shape[1]
    d2 = W2_mu.shape[1]
    grid = n_rows // block_rows
    return pl.pallas_call(
        _vb_body,
        grid=(grid,),
        in_specs=[
            pl.BlockSpec((block_rows, d0), lambda i: (row_block0 + i, 0)),
            pl.BlockSpec((d0, d1), lambda i: (0, 0)),
            pl.BlockSpec((d0, d1), lambda i: (0, 0)),
            pl.BlockSpec((d1, d2), lambda i: (0, 0)),
            pl.BlockSpec((d1, d2), lambda i: (0, 0)),
        ],
        out_specs=pl.BlockSpec((block_rows, 2 * d2), lambda i: (i, 0)),
        out_shape=jax.ShapeDtypeStruct((n_rows, 2 * d2), jnp.float32),
    )(X, W1_mu, W1_logsig2, W2_mu, W2_logsig2)


def _make_seg_sum(n, num_seg_pad, width, chunk):
    info = plsc.get_sparse_core_info()
    nc, ns = info.num_cores, info.num_subcores
    nw = nc * ns
    rows_per_tile = n // nw
    n_chunks = rows_per_tile // chunk
    segs_per_tile = num_seg_pad // ns

    mesh = plsc.VectorSubcoreMesh(core_axis_name="c", subcore_axis_name="s")

    @functools.partial(
        pl.kernel,
        out_type=jax.ShapeDtypeStruct((nc, num_seg_pad, width), jnp.float32),
        mesh=mesh,
        scratch_types=[
            pltpu.VMEM((n_chunks * chunk,), jnp.int32),
            pltpu.VMEM((chunk, width), jnp.float32),
            pltpu.VMEM((chunk, width), jnp.float32),
            pltpu.VMEM((80, width), jnp.float32),
            pltpu.VMEM_SHARED((num_seg_pad, width), jnp.float32),
            pltpu.SemaphoreType.DMA,
            pltpu.SemaphoreType.DMA,
        ],
    )
    def seg_sum(pw_hbm, idx_hbm, part_hbm,
                idx_v, buf0, buf1, zbuf, acc_sh, sem0, sem1):
        cid = lax.axis_index("c")
        sid = lax.axis_index("s")
        wid = sid * nc + cid
        base = wid * rows_per_tile
        bufs = (buf0, buf1)
        sems = (sem0, sem1)

        def issue(g, b):
            pltpu.async_copy(pw_hbm.at[pl.ds(base + g * chunk, chunk)],
                             bufs[b], sems[b])

        for b in range(2):
            issue(b, b)

        pltpu.sync_copy(idx_hbm.at[wid], idx_v)

        z16 = jnp.zeros((16,), jnp.float32)

        def zrow(i, c):
            for j in range(width // 16):
                zbuf[i, pl.ds(j * 16, 16)] = z16
            return c

        lax.fori_loop(0, 80, zrow, 0)
        for k in range(segs_per_tile // 80):
            pltpu.sync_copy(
                zbuf,
                acc_sh.at[pl.ds(sid * segs_per_tile + k * 80, 80)])
        plsc.subcore_barrier()


        def body(j, carry):
            for b in range(2):
                g = 2 * j + b
                pltpu.make_async_copy(pw_hbm.at[pl.ds(0, chunk)],
                                      bufs[b], sems[b]).wait()
                pltpu.sync_copy(bufs[b],
                                acc_sh.at[idx_v.at[pl.ds(g * chunk, chunk)]],
                                add=True)
                issue(jnp.minimum(g + 2, n_chunks - 1), b)
            return carry

        lax.fori_loop(0, n_chunks // 2, body, 0)

        if n_chunks % 2:
            gl = n_chunks - 1
            pltpu.make_async_copy(pw_hbm.at[pl.ds(0, chunk)],
                                  bufs[0], sems[0]).wait()
            pltpu.sync_copy(bufs[0],
                            acc_sh.at[idx_v.at[pl.ds(gl * chunk, chunk)]],
                            add=True)
            pltpu.make_async_copy(pw_hbm.at[pl.ds(0, chunk)],
                                  bufs[1], sems[1]).wait()
        else:
            pltpu.make_async_copy(pw_hbm.at[pl.ds(0, chunk)],
                                  bufs[0], sems[0]).wait()
            pltpu.make_async_copy(pw_hbm.at[pl.ds(0, chunk)],
                                  bufs[1], sems[1]).wait()
        plsc.subcore_barrier()

        pltpu.sync_copy(
            acc_sh.at[pl.ds(sid * segs_per_tile, segs_per_tile)],
            part_hbm.at[cid, pl.ds(sid * segs_per_tile, segs_per_tile)],
        )

    return seg_sum


def _fin_body(*refs):
    means_ref, vars_ref = refs[-2], refs[-1]
    d2 = means_ref.shape[1]
    s = refs[0][0] + refs[0][1]
    for r in refs[1:-2]:
        s = s + r[0] + r[1]
    w = s[:, :d2] + EPS
    var = 1.0 / w
    means_ref[...] = s[:, d2:] * var
    vars_ref[...] = var


def _finalize(parts, num_seg_pad, d2):
    width = 2 * d2
    bs = 1280
    grid = num_seg_pad // bs
    return pl.pallas_call(
        _fin_body,
        grid=(grid,),
        in_specs=[pl.BlockSpec((2, bs, width), lambda i: (0, i, 0))
                  for _ in parts],
        out_specs=(pl.BlockSpec((bs, d2), lambda i: (i, 0)),
                   pl.BlockSpec((bs, d2), lambda i: (i, 0))),
        out_shape=(
            jax.ShapeDtypeStruct((num_seg_pad, d2), jnp.float32),
            jax.ShapeDtypeStruct((num_seg_pad, d2), jnp.float32),
        ),
    )(*parts)


def kernel(X, W1_mu, W1_logsig2, W2_mu, W2_logsig2, X_idx):
    n = X.shape[0]
    d2 = W2_mu.shape[1]
    num_seg = 10000
    num_seg_pad = 10240
    width = 2 * d2

    nchunk = 5
    chunk = 80
    block_rows = 2000
    info = plsc.get_sparse_core_info()
    nw = info.num_cores * info.num_subcores
    n_c = n // nchunk
    n_chunks = n_c // (nw * chunk)
    assert n_chunks >= 2 and n_c % (nw * chunk) == 0
    assert n_c % block_rows == 0

    idx3d = X_idx.reshape(nchunk, nw, n_chunks * chunk)
    seg_sum = _make_seg_sum(n_c, num_seg_pad, width, chunk=chunk)

    parts = []
    for c in range(nchunk):
        pw = _vb_layers(X, W1_mu, W1_logsig2, W2_mu, W2_logsig2,
                        block_rows=block_rows,
                        row_block0=c * (n_c // block_rows), n_rows=n_c)
        parts.append(seg_sum(pw, idx3d[c]))

    means_p, vars_p = _finalize(parts, num_seg_pad, d2)
    return means_p[:num_seg], vars_p[:num_seg]


# speedup vs baseline: 1.3670x; 1.1726x over previous
import functools

import jax
import jax.numpy as jnp
from jax import lax
from jax.experimental import pallas as pl
from jax.experimental.pallas import tpu as pltpu
from jax.experimental.pallas import tpu_sc as plsc

EPS = 1e-8


def _vb_body(x_ref, w1mu_ref, w1ls_ref, w2mu_ref, w2ls_ref, out_ref):
    x = x_ref[...]
    d1 = w1mu_ref.shape[1]
    d2 = w2mu_ref.shape[1]
    scale = (2.0 / d1) ** 0.5
    w1mu = scale * w1mu_ref[...]
    sig21 = (scale * scale) * jnp.exp(w1ls_ref[...])
    w2mu = w2mu_ref[...]
    sig22 = jnp.exp(w2ls_ref[...])

    dot = functools.partial(jnp.dot, preferred_element_type=jnp.float32)
    m1 = jnp.maximum(dot(x, w1mu), 0.0)
    v1 = dot(x * x, sig21)
    m2 = dot(m1, w2mu)
    v2 = dot(m1 * m1 + v1, sig22) + dot(v1, w2mu * w2mu)

    p = 1.0 / (v2 + EPS)
    out_ref[:, :d2] = p
    out_ref[:, d2:] = p * m2


def _vb_layers(X, W1_mu, W1_logsig2, W2_mu, W2_logsig2, block_rows,
               row_block0, n_rows):
    d0 = X.shape[1]
    d1 = W1_mu.shape[1]
    d2 = W2_mu.shape[1]
    grid = n_rows // block_rows
    return pl.pallas_call(
        _vb_body,
        grid=(grid,),
        in_specs=[
            pl.BlockSpec((block_rows, d0), lambda i: (row_block0 + i, 0)),
            pl.BlockSpec((d0, d1), lambda i: (0, 0)),
            pl.BlockSpec((d0, d1), lambda i: (0, 0)),
            pl.BlockSpec((d1, d2), lambda i: (0, 0)),
            pl.BlockSpec((d1, d2), lambda i: (0, 0)),
        ],
        out_specs=pl.BlockSpec((block_rows, 2 * d2), lambda i: (i, 0)),
        out_shape=jax.ShapeDtypeStruct((n_rows, 2 * d2), jnp.float32),
    )(X, W1_mu, W1_logsig2, W2_mu, W2_logsig2)


def _make_seg_sum(n, num_seg_pad, width, chunk):
    info = plsc.get_sparse_core_info()
    nc, ns = info.num_cores, info.num_subcores
    nw = nc * ns
    rows_per_tile = n // nw
    n_chunks = rows_per_tile // chunk
    segs_per_tile = num_seg_pad // ns

    mesh = plsc.VectorSubcoreMesh(core_axis_name="c", subcore_axis_name="s")

    @functools.partial(
        pl.kernel,
        out_type=jax.ShapeDtypeStruct((nc, num_seg_pad, width), jnp.float32),
        mesh=mesh,
        scratch_types=[
            pltpu.VMEM((n_chunks * chunk,), jnp.int32),
            pltpu.VMEM((chunk, width), jnp.float32),
            pltpu.VMEM((chunk, width), jnp.float32),
            pltpu.VMEM((80, width), jnp.float32),
            pltpu.VMEM_SHARED((num_seg_pad, width), jnp.float32),
            pltpu.SemaphoreType.DMA,
            pltpu.SemaphoreType.DMA,
        ],
    )
    def seg_sum(pw_hbm, idx_hbm, part_hbm,
                idx_v, buf0, buf1, zbuf, acc_sh, sem0, sem1):
        cid = lax.axis_index("c")
        sid = lax.axis_index("s")
        wid = sid * nc + cid
        base = wid * rows_per_tile
        bufs = (buf0, buf1)
        sems = (sem0, sem1)

        def issue(g, b):
            pltpu.async_copy(pw_hbm.at[pl.ds(base + g * chunk, chunk)],
                             bufs[b], sems[b])

        for b in range(2):
            issue(b, b)

        pltpu.sync_copy(idx_hbm.at[wid], idx_v)

        z16 = jnp.zeros((16,), jnp.float32)

        def zrow(i, c):
            for j in range(width // 16):
                zbuf[i, pl.ds(j * 16, 16)] = z16
            return c

        lax.fori_loop(0, 80, zrow, 0)
        for k in range(segs_per_tile // 80):
            pltpu.sync_copy(
                zbuf,
                acc_sh.at[pl.ds(sid * segs_per_tile + k * 80, 80)])
        plsc.subcore_barrier()


        def body(j, carry):
            for b in range(2):
                g = 2 * j + b
                pltpu.make_async_copy(pw_hbm.at[pl.ds(0, chunk)],
                                      bufs[b], sems[b]).wait()
                pltpu.sync_copy(bufs[b],
                                acc_sh.at[idx_v.at[pl.ds(g * chunk, chunk)]],
                                add=True)
                issue(jnp.minimum(g + 2, n_chunks - 1), b)
            return carry

        lax.fori_loop(0, n_chunks // 2, body, 0)

        if n_chunks % 2:
            gl = n_chunks - 1
            pltpu.make_async_copy(pw_hbm.at[pl.ds(0, chunk)],
                                  bufs[0], sems[0]).wait()
            pltpu.sync_copy(bufs[0],
                            acc_sh.at[idx_v.at[pl.ds(gl * chunk, chunk)]],
                            add=True)
            pltpu.make_async_copy(pw_hbm.at[pl.ds(0, chunk)],
                                  bufs[1], sems[1]).wait()
        else:
            pltpu.make_async_copy(pw_hbm.at[pl.ds(0, chunk)],
                                  bufs[0], sems[0]).wait()
            pltpu.make_async_copy(pw_hbm.at[pl.ds(0, chunk)],
                                  bufs[1], sems[1]).wait()
        plsc.subcore_barrier()

        pltpu.sync_copy(
            acc_sh.at[pl.ds(sid * segs_per_tile, segs_per_tile)],
            part_hbm.at[cid, pl.ds(sid * segs_per_tile, segs_per_tile)],
        )

    return seg_sum


def _fin_body(*refs):
    means_ref, vars_ref = refs[-2], refs[-1]
    d2 = means_ref.shape[1]
    s = refs[0][0] + refs[0][1]
    for r in refs[1:-2]:
        s = s + r[0] + r[1]
    w = s[:, :d2] + EPS
    var = 1.0 / w
    means_ref[...] = s[:, d2:] * var
    vars_ref[...] = var


def _finalize(parts, num_seg_pad, d2):
    width = 2 * d2
    bs = 1280
    grid = num_seg_pad // bs
    return pl.pallas_call(
        _fin_body,
        grid=(grid,),
        in_specs=[pl.BlockSpec((2, bs, width), lambda i: (0, i, 0))
                  for _ in parts],
        out_specs=(pl.BlockSpec((bs, d2), lambda i: (i, 0)),
                   pl.BlockSpec((bs, d2), lambda i: (i, 0))),
        out_shape=(
            jax.ShapeDtypeStruct((num_seg_pad, d2), jnp.float32),
            jax.ShapeDtypeStruct((num_seg_pad, d2), jnp.float32),
        ),
    )(*parts)


def kernel(X, W1_mu, W1_logsig2, W2_mu, W2_logsig2, X_idx):
    n = X.shape[0]
    d2 = W2_mu.shape[1]
    num_seg = 10000
    num_seg_pad = 10240
    width = 2 * d2

    nchunk = 5
    chunk = 80
    block_rows = 8000
    info = plsc.get_sparse_core_info()
    nw = info.num_cores * info.num_subcores
    n_c = n // nchunk
    n_chunks = n_c // (nw * chunk)
    assert n_chunks >= 2 and n_c % (nw * chunk) == 0
    assert n_c % block_rows == 0

    idx3d = X_idx.reshape(nchunk, nw, n_chunks * chunk)
    seg_sum = _make_seg_sum(n_c, num_seg_pad, width, chunk=chunk)

    parts = []
    for c in range(nchunk):
        pw = _vb_layers(X, W1_mu, W1_logsig2, W2_mu, W2_logsig2,
                        block_rows=block_rows,
                        row_block0=c * (n_c // block_rows), n_rows=n_c)
        parts.append(seg_sum(pw, idx3d[c]))

    means_p, vars_p = _finalize(parts, num_seg_pad, d2)
    return means_p[:num_seg], vars_p[:num_seg]


# speedup vs baseline: 1.3672x; 1.0001x over previous
import functools

import jax
import jax.numpy as jnp
from jax import lax
from jax.experimental import pallas as pl
from jax.experimental.pallas import tpu as pltpu
from jax.experimental.pallas import tpu_sc as plsc

EPS = 1e-8


def _vb_body(x_ref, w1mu_ref, w1ls_ref, w2mu_ref, w2ls_ref, out_ref):
    x = x_ref[...]
    d1 = w1mu_ref.shape[1]
    d2 = w2mu_ref.shape[1]
    scale = (2.0 / d1) ** 0.5
    w1mu = scale * w1mu_ref[...]
    sig21 = (scale * scale) * jnp.exp(w1ls_ref[...])
    w2mu = w2mu_ref[...]
    sig22 = jnp.exp(w2ls_ref[...])

    dot = functools.partial(jnp.dot, preferred_element_type=jnp.float32)
    m1 = jnp.maximum(dot(x, w1mu), 0.0)
    v1 = dot(x * x, sig21)
    m2 = dot(m1, w2mu)
    v2 = dot(m1 * m1 + v1, sig22) + dot(v1, w2mu * w2mu)

    p = 1.0 / (v2 + EPS)
    out_ref[:, :d2] = p
    out_ref[:, d2:] = p * m2


def _vb_layers(X, W1_mu, W1_logsig2, W2_mu, W2_logsig2, block_rows,
               row_block0, n_rows):
    d0 = X.shape[1]
    d1 = W1_mu.shape[1]
    d2 = W2_mu.shape[1]
    grid = n_rows // block_rows
    return pl.pallas_call(
        _vb_body,
        grid=(grid,),
        in_specs=[
            pl.BlockSpec((block_rows, d0), lambda i: (row_block0 + i, 0)),
            pl.BlockSpec((d0, d1), lambda i: (0, 0)),
            pl.BlockSpec((d0, d1), lambda i: (0, 0)),
            pl.BlockSpec((d1, d2), lambda i: (0, 0)),
            pl.BlockSpec((d1, d2), lambda i: (0, 0)),
        ],
        out_specs=pl.BlockSpec((block_rows, 2 * d2), lambda i: (i, 0)),
        out_shape=jax.ShapeDtypeStruct((n_rows, 2 * d2), jnp.float32),
    )(X, W1_mu, W1_logsig2, W2_mu, W2_logsig2)


def _make_seg_sum(n, num_seg_pad, width, chunk):
    info = plsc.get_sparse_core_info()
    nc, ns = info.num_cores, info.num_subcores
    nw = nc * ns
    rows_per_tile = n // nw
    n_chunks = rows_per_tile // chunk
    segs_per_tile = num_seg_pad // ns

    mesh = plsc.VectorSubcoreMesh(core_axis_name="c", subcore_axis_name="s")

    @functools.partial(
        pl.kernel,
        out_type=jax.ShapeDtypeStruct((nc, num_seg_pad, width), jnp.float32),
        mesh=mesh,
        scratch_types=[
            pltpu.VMEM((n_chunks * chunk,), jnp.int32),
            pltpu.VMEM((chunk, width), jnp.float32),
            pltpu.VMEM((chunk, width), jnp.float32),
            pltpu.VMEM((80, width), jnp.float32),
            pltpu.VMEM_SHARED((num_seg_pad, width), jnp.float32),
            pltpu.SemaphoreType.DMA,
            pltpu.SemaphoreType.DMA,
        ],
    )
    def seg_sum(pw_hbm, idx_hbm, part_hbm,
                idx_v, buf0, buf1, zbuf, acc_sh, sem0, sem1):
        cid = lax.axis_index("c")
        sid = lax.axis_index("s")
        wid = sid * nc + cid
        base = wid * rows_per_tile
        bufs = (buf0, buf1)
        sems = (sem0, sem1)

        def issue(g, b):
            pltpu.async_copy(pw_hbm.at[pl.ds(base + g * chunk, chunk)],
                             bufs[b], sems[b])

        for b in range(2):
            issue(b, b)

        pltpu.sync_copy(idx_hbm.at[wid], idx_v)

        z16 = jnp.zeros((16,), jnp.float32)

        def zrow(i, c):
            for j in range(width // 16):
                zbuf[i, pl.ds(j * 16, 16)] = z16
            return c

        lax.fori_loop(0, 80, zrow, 0)
        for k in range(segs_per_tile // 80):
            pltpu.sync_copy(
                zbuf,
                acc_sh.at[pl.ds(sid * segs_per_tile + k * 80, 80)])
        plsc.subcore_barrier()


        def body(j, carry):
            for b in range(2):
                g = 2 * j + b
                pltpu.make_async_copy(pw_hbm.at[pl.ds(0, chunk)],
                                      bufs[b], sems[b]).wait()
                pltpu.sync_copy(bufs[b],
                                acc_sh.at[idx_v.at[pl.ds(g * chunk, chunk)]],
                                add=True)
                issue(jnp.minimum(g + 2, n_chunks - 1), b)
            return carry

        lax.fori_loop(0, n_chunks // 2, body, 0)

        if n_chunks % 2:
            gl = n_chunks - 1
            pltpu.make_async_copy(pw_hbm.at[pl.ds(0, chunk)],
                                  bufs[0], sems[0]).wait()
            pltpu.sync_copy(bufs[0],
                            acc_sh.at[idx_v.at[pl.ds(gl * chunk, chunk)]],
                            add=True)
            pltpu.make_async_copy(pw_hbm.at[pl.ds(0, chunk)],
                                  bufs[1], sems[1]).wait()
        else:
            pltpu.make_async_copy(pw_hbm.at[pl.ds(0, chunk)],
                                  bufs[0], sems[0]).wait()
            pltpu.make_async_copy(pw_hbm.at[pl.ds(0, chunk)],
                                  bufs[1], sems[1]).wait()
        plsc.subcore_barrier()

        pltpu.sync_copy(
            acc_sh.at[pl.ds(sid * segs_per_tile, segs_per_tile)],
            part_hbm.at[cid, pl.ds(sid * segs_per_tile, segs_per_tile)],
        )

    return seg_sum


def _fin_body(*refs):
    means_ref, vars_ref = refs[-2], refs[-1]
    d2 = means_ref.shape[1]
    s = refs[0][0] + refs[0][1]
    for r in refs[1:-2]:
        s = s + r[0] + r[1]
    w = s[:, :d2] + EPS
    var = 1.0 / w
    means_ref[...] = s[:, d2:] * var
    vars_ref[...] = var


def _finalize(parts, num_seg_pad, d2):
    width = 2 * d2
    bs = 1280
    grid = num_seg_pad // bs
    return pl.pallas_call(
        _fin_body,
        grid=(grid,),
        in_specs=[pl.BlockSpec((2, bs, width), lambda i: (0, i, 0))
                  for _ in parts],
        out_specs=(pl.BlockSpec((bs, d2), lambda i: (i, 0)),
                   pl.BlockSpec((bs, d2), lambda i: (i, 0))),
        out_shape=(
            jax.ShapeDtypeStruct((num_seg_pad, d2), jnp.float32),
            jax.ShapeDtypeStruct((num_seg_pad, d2), jnp.float32),
        ),
    )(*parts)


def kernel(X, W1_mu, W1_logsig2, W2_mu, W2_logsig2, X_idx):
    n = X.shape[0]
    d2 = W2_mu.shape[1]
    num_seg = 10000
    num_seg_pad = 10240
    width = 2 * d2

    nchunk = 5
    chunk = 80
    block_rows = 16000
    info = plsc.get_sparse_core_info()
    nw = info.num_cores * info.num_subcores
    n_c = n // nchunk
    n_chunks = n_c // (nw * chunk)
    assert n_chunks >= 2 and n_c % (nw * chunk) == 0
    assert n_c % block_rows == 0

    idx3d = X_idx.reshape(nchunk, nw, n_chunks * chunk)
    seg_sum = _make_seg_sum(n_c, num_seg_pad, width, chunk=chunk)

    parts = []
    for c in range(nchunk):
        pw = _vb_layers(X, W1_mu, W1_logsig2, W2_mu, W2_logsig2,
                        block_rows=block_rows,
                        row_block0=c * (n_c // block_rows), n_rows=n_c)
        parts.append(seg_sum(pw, idx3d[c]))

    means_p, vars_p = _finalize(parts, num_seg_pad, d2)
    return means_p[:num_seg], vars_p[:num_seg]


# speedup vs baseline: 1.3693x; 1.0016x over previous
import functools

import jax
import jax.numpy as jnp
from jax import lax
from jax.experimental import pallas as pl
from jax.experimental.pallas import tpu as pltpu
from jax.experimental.pallas import tpu_sc as plsc

EPS = 1e-8


def _vb_body(x_ref, w1mu_ref, w1ls_ref, w2mu_ref, w2ls_ref, out_ref):
    x = x_ref[...]
    d1 = w1mu_ref.shape[1]
    d2 = w2mu_ref.shape[1]
    scale = (2.0 / d1) ** 0.5
    w1mu = scale * w1mu_ref[...]
    sig21 = (scale * scale) * jnp.exp(w1ls_ref[...])
    w2mu = w2mu_ref[...]
    sig22 = jnp.exp(w2ls_ref[...])

    dot = functools.partial(jnp.dot, preferred_element_type=jnp.float32)
    m1 = jnp.maximum(dot(x, w1mu), 0.0)
    v1 = dot(x * x, sig21)
    m2 = dot(m1, w2mu)
    v2 = dot(m1 * m1 + v1, sig22) + dot(v1, w2mu * w2mu)

    p = 1.0 / (v2 + EPS)
    out_ref[:, :d2] = p
    out_ref[:, d2:] = p * m2


def _vb_layers(X, W1_mu, W1_logsig2, W2_mu, W2_logsig2, block_rows,
               row_block0, n_rows):
    d0 = X.shape[1]
    d1 = W1_mu.shape[1]
    d2 = W2_mu.shape[1]
    grid = n_rows // block_rows
    return pl.pallas_call(
        _vb_body,
        grid=(grid,),
        in_specs=[
            pl.BlockSpec((block_rows, d0), lambda i: (row_block0 + i, 0)),
            pl.BlockSpec((d0, d1), lambda i: (0, 0)),
            pl.BlockSpec((d0, d1), lambda i: (0, 0)),
            pl.BlockSpec((d1, d2), lambda i: (0, 0)),
            pl.BlockSpec((d1, d2), lambda i: (0, 0)),
        ],
        out_specs=pl.BlockSpec((block_rows, 2 * d2), lambda i: (i, 0)),
        out_shape=jax.ShapeDtypeStruct((n_rows, 2 * d2), jnp.float32),
    )(X, W1_mu, W1_logsig2, W2_mu, W2_logsig2)


def _make_seg_sum(n, num_seg_pad, width, chunk):
    info = plsc.get_sparse_core_info()
    nc, ns = info.num_cores, info.num_subcores
    nw = nc * ns
    rows_per_tile = n // nw
    n_chunks = rows_per_tile // chunk
    segs_per_tile = num_seg_pad // ns

    mesh = plsc.VectorSubcoreMesh(core_axis_name="c", subcore_axis_name="s")

    @functools.partial(
        pl.kernel,
        out_type=jax.ShapeDtypeStruct((nc, num_seg_pad, width), jnp.float32),
        mesh=mesh,
        scratch_types=[
            pltpu.VMEM((n_chunks * chunk,), jnp.int32),
            pltpu.VMEM((chunk, width), jnp.float32),
            pltpu.VMEM((chunk, width), jnp.float32),
            pltpu.VMEM((80, width), jnp.float32),
            pltpu.VMEM_SHARED((num_seg_pad, width), jnp.float32),
            pltpu.SemaphoreType.DMA,
            pltpu.SemaphoreType.DMA,
        ],
    )
    def seg_sum(pw_hbm, idx_hbm, part_hbm,
                idx_v, buf0, buf1, zbuf, acc_sh, sem0, sem1):
        cid = lax.axis_index("c")
        sid = lax.axis_index("s")
        wid = sid * nc + cid
        base = wid * rows_per_tile
        bufs = (buf0, buf1)
        sems = (sem0, sem1)

        def issue(g, b):
            pltpu.async_copy(pw_hbm.at[pl.ds(base + g * chunk, chunk)],
                             bufs[b], sems[b])

        for b in range(2):
            issue(b, b)

        pltpu.sync_copy(idx_hbm.at[wid], idx_v)

        z16 = jnp.zeros((16,), jnp.float32)

        def zrow(i, c):
            for j in range(width // 16):
                zbuf[i, pl.ds(j * 16, 16)] = z16
            return c

        lax.fori_loop(0, 80, zrow, 0)
        for k in range(segs_per_tile // 80):
            pltpu.sync_copy(
                zbuf,
                acc_sh.at[pl.ds(sid * segs_per_tile + k * 80, 80)])
        plsc.subcore_barrier()


        def body(j, carry):
            for b in range(2):
                g = 2 * j + b
                pltpu.make_async_copy(pw_hbm.at[pl.ds(0, chunk)],
                                      bufs[b], sems[b]).wait()
                pltpu.sync_copy(bufs[b],
                                acc_sh.at[idx_v.at[pl.ds(g * chunk, chunk)]],
                                add=True)
                issue(jnp.minimum(g + 2, n_chunks - 1), b)
            return carry

        lax.fori_loop(0, n_chunks // 2, body, 0)

        if n_chunks % 2:
            gl = n_chunks - 1
            pltpu.make_async_copy(pw_hbm.at[pl.ds(0, chunk)],
                                  bufs[0], sems[0]).wait()
            pltpu.sync_copy(bufs[0],
                            acc_sh.at[idx_v.at[pl.ds(gl * chunk, chunk)]],
                            add=True)
            pltpu.make_async_copy(pw_hbm.at[pl.ds(0, chunk)],
                                  bufs[1], sems[1]).wait()
        else:
            pltpu.make_async_copy(pw_hbm.at[pl.ds(0, chunk)],
                                  bufs[0], sems[0]).wait()
            pltpu.make_async_copy(pw_hbm.at[pl.ds(0, chunk)],
                                  bufs[1], sems[1]).wait()
        plsc.subcore_barrier()

        pltpu.sync_copy(
            acc_sh.at[pl.ds(sid * segs_per_tile, segs_per_tile)],
            part_hbm.at[cid, pl.ds(sid * segs_per_tile, segs_per_tile)],
        )

    return seg_sum


def _fin_body(*refs):
    means_ref, vars_ref = refs[-2], refs[-1]
    d2 = means_ref.shape[1]
    s = refs[0][0] + refs[0][1]
    for r in refs[1:-2]:
        s = s + r[0] + r[1]
    w = s[:, :d2] + EPS
    var = 1.0 / w
    means_ref[...] = s[:, d2:] * var
    vars_ref[...] = var


def _finalize(parts, num_seg_pad, d2):
    width = 2 * d2
    bs = 1280
    grid = num_seg_pad // bs
    return pl.pallas_call(
        _fin_body,
        grid=(grid,),
        in_specs=[pl.BlockSpec((2, bs, width), lambda i: (0, i, 0))
                  for _ in parts],
        out_specs=(pl.BlockSpec((bs, d2), lambda i: (i, 0)),
                   pl.BlockSpec((bs, d2), lambda i: (i, 0))),
        out_shape=(
            jax.ShapeDtypeStruct((num_seg_pad, d2), jnp.float32),
            jax.ShapeDtypeStruct((num_seg_pad, d2), jnp.float32),
        ),
    )(*parts)


def kernel(X, W1_mu, W1_logsig2, W2_mu, W2_logsig2, X_idx):
    n = X.shape[0]
    d2 = W2_mu.shape[1]
    num_seg = 10000
    num_seg_pad = 10240
    width = 2 * d2

    nchunk = 5
    chunk = 80
    block_rows = 8000
    info = plsc.get_sparse_core_info()
    nw = info.num_cores * info.num_subcores
    n_c = n // nchunk
    n_chunks = n_c // (nw * chunk)
    assert n_chunks >= 2 and n_c % (nw * chunk) == 0
    assert n_c % block_rows == 0

    idx3d = X_idx.reshape(nchunk, nw, n_chunks * chunk)
    seg_sum = _make_seg_sum(n_c, num_seg_pad, width, chunk=chunk)

    parts = []
    for c in range(nchunk):
        pw = _vb_layers(X, W1_mu, W1_logsig2, W2_mu, W2_logsig2,
                        block_rows=block_rows,
                        row_block0=c * (n_c // block_rows), n_rows=n_c)
        parts.append(seg_sum(pw, idx3d[c]))

    means_p, vars_p = _finalize(parts, num_seg_pad, d2)
    return means_p[:num_seg], vars_p[:num_seg]
